# baseline (speedup 1.0000x reference)
import functools

import jax
import jax.numpy as jnp
from jax import lax
from jax.experimental import pallas as pl
from jax.experimental.pallas import tpu as pltpu
from jax.experimental.pallas import tpu_sc as plsc

BATCH = 16384
HIST = 50
EMBED_DIM = 32
LANES = 16

_info = plsc.get_sparse_core_info()
NUM_CORES = _info.num_cores
NUM_SUBCORES = _info.num_subcores
NUM_WORKERS = NUM_CORES * NUM_SUBCORES
BPW = BATCH // NUM_WORKERS
NGROUPS = BPW // LANES

_mesh = plsc.VectorSubcoreMesh(core_axis_name="c", subcore_axis_name="s")


@functools.partial(
    pl.kernel,
    mesh=_mesh,
    out_type=jax.ShapeDtypeStruct((HIST, EMBED_DIM, BATCH), jnp.float32),
    scratch_types=[
        pltpu.VMEM((HIST, BPW), jnp.int32),
        pltpu.VMEM((2, BPW, EMBED_DIM), jnp.float32),
        pltpu.VMEM((2, EMBED_DIM, BPW), jnp.float32),
        pltpu.SemaphoreType.DMA,
        pltpu.SemaphoreType.DMA,
    ],
    compiler_params=pltpu.CompilerParams(
        use_tc_tiling_on_sc=False,
        needs_layout_passes=False,
        disable_bounds_checks=True,
    ),
)
def _gather_t(xt_hbm, table_hbm, out_hbm, idx_all, rows_v, tr_v, gsem, osem):
    wid = lax.axis_index("s") * NUM_CORES + lax.axis_index("c")
    b0 = wid * BPW

    iota = lax.iota(jnp.int32, LANES)
    group_rows = [iota + (g * LANES) for g in range(NGROUPS)]

    pltpu.sync_copy(xt_hbm.at[:, pl.ds(b0, BPW)], idx_all)

    def start_gather(h, slot):
        pltpu.async_copy(table_hbm.at[idx_all.at[h]], rows_v.at[slot], gsem)

    def wait_gather():
        pltpu.make_async_copy(
            table_hbm.at[pl.ds(0, BPW)], rows_v.at[0], gsem
        ).wait()

    def start_write(h, slot):
        pltpu.async_copy(tr_v.at[slot], out_hbm.at[h, :, pl.ds(b0, BPW)], osem)

    def wait_write():
        pltpu.make_async_copy(
            tr_v.at[0], out_hbm.at[0, :, pl.ds(b0, BPW)], osem
        ).wait()

    def transpose_slow(slot):
        def d_body(d, c):
            cols = jnp.full((LANES,), 0, jnp.int32) + d
            for g in range(NGROUPS):
                vals = plsc.load_gather(rows_v.at[slot], [group_rows[g], cols])
                tr_v[slot, d, pl.ds(g * LANES, LANES)] = vals
            return c

        lax.fori_loop(0, EMBED_DIM, d_body, 0)

    def transpose_fast(slot):
        slot_splat = jnp.full((LANES,), 0, jnp.int32) + slot
        for d in range(EMBED_DIM):
            cols = jnp.full((LANES,), d, jnp.int32)
            for g in range(NGROUPS):
                vals = plsc.load_gather(
                    rows_v, [slot_splat, group_rows[g], cols]
                )
                tr_v[slot, d, pl.ds(g * LANES, LANES)] = vals

    start_gather(0, 0)
    start_gather(1, 1)
    wait_gather()
    transpose_slow(0)
    start_write(0, 0)
    start_gather(2, 0)
    wait_gather()
    transpose_slow(1)
    start_write(1, 1)
    start_gather(3, 1)

    def h_body(h, c):
        slot = h % 2
        wait_gather()
        wait_write()
        transpose_fast(slot)
        start_write(h, slot)
        start_gather(jnp.minimum(h + 2, HIST - 1), slot)
        return c

    lax.fori_loop(2, HIST, h_body, 0)
    wait_gather()
    wait_gather()
    wait_write()
    wait_write()


def kernel(x, table):
    xt = x.astype(jnp.int32).T
    out_t = _gather_t(xt, table)
    return out_t.transpose(2, 0, 1)


# speedup vs baseline: 1.4958x; 1.4958x over previous
import functools

import jax
import jax.numpy as jnp
from jax import lax
from jax.experimental import pallas as pl
from jax.experimental.pallas import tpu as pltpu
from jax.experimental.pallas import tpu_sc as plsc

BATCH = 16384
HIST = 50
EMBED_DIM = 32
LANES = 16

_info = plsc.get_sparse_core_info()
NUM_CORES = _info.num_cores
NUM_SUBCORES = _info.num_subcores
NUM_WORKERS = NUM_CORES * NUM_SUBCORES
BPW = BATCH // NUM_WORKERS
NGROUPS = BPW // LANES

_mesh = plsc.VectorSubcoreMesh(core_axis_name="c", subcore_axis_name="s")


@functools.partial(
    pl.kernel,
    mesh=_mesh,
    out_type=jax.ShapeDtypeStruct((HIST, EMBED_DIM, BATCH), jnp.float32),
    scratch_types=[
        pltpu.VMEM((HIST, BPW), jnp.int32),
        pltpu.VMEM((2, BPW, EMBED_DIM), jnp.float32),
        pltpu.VMEM((2, EMBED_DIM, BPW), jnp.float32),
        pltpu.SemaphoreType.DMA,
        pltpu.SemaphoreType.DMA,
    ],
    compiler_params=pltpu.CompilerParams(
        use_tc_tiling_on_sc=False,
        needs_layout_passes=False,
        disable_bounds_checks=True,
    ),
)
def _gather_t(xt_hbm, table_hbm, out_hbm, idx_all, rows_v, tr_v, gsem, osem):
    wid = lax.axis_index("s") * NUM_CORES + lax.axis_index("c")
    b0 = wid * BPW

    iota = lax.iota(jnp.int32, LANES)
    group_rows = [iota + (g * LANES) for g in range(NGROUPS)]

    pltpu.sync_copy(xt_hbm.at[:, pl.ds(b0, BPW)], idx_all)

    def start_gather(h, slot):
        pltpu.async_copy(table_hbm.at[idx_all.at[h]], rows_v.at[slot], gsem)

    def wait_gather():
        pltpu.make_async_copy(
            table_hbm.at[pl.ds(0, BPW)], rows_v.at[0], gsem
        ).wait()

    def start_write(h, slot):
        pltpu.async_copy(tr_v.at[slot], out_hbm.at[h, :, pl.ds(b0, BPW)], osem)

    def wait_write():
        pltpu.make_async_copy(
            tr_v.at[0], out_hbm.at[0, :, pl.ds(b0, BPW)], osem
        ).wait()

    def transpose_block(slot):
        slot_splat = jnp.full((LANES,), 0, jnp.int32) + slot

        def d_body(d, c):
            diag = (iota + d) & (EMBED_DIM - 1)
            for g in range(NGROUPS):
                vals = plsc.load_gather(rows_v, [slot_splat, group_rows[g], diag])
                plsc.store_scatter(tr_v, [slot_splat, diag, group_rows[g]], vals)
            return c

        lax.fori_loop(0, EMBED_DIM, d_body, 0)


    start_gather(0, 0)
    start_gather(1, 1)
    wait_gather()
    transpose_block(0)
    start_write(0, 0)
    start_gather(2, 0)
    wait_gather()
    transpose_block(1)
    start_write(1, 1)
    start_gather(3, 1)

    def h_body(h, c):
        slot = h % 2
        wait_gather()
        wait_write()
        transpose_block(slot)
        start_write(h, slot)
        start_gather(jnp.minimum(h + 2, HIST - 1), slot)
        return c

    lax.fori_loop(2, HIST, h_body, 0)
    wait_gather()
    wait_gather()
    wait_write()
    wait_write()


def kernel(x, table):
    xt = x.astype(jnp.int32).T
    out_t = _gather_t(xt, table)
    return out_t.transpose(2, 0, 1)
